# baseline (speedup 1.0000x reference)
import functools

import jax
import jax.numpy as jnp
from jax import lax
from jax.experimental import pallas as pl
from jax.experimental.pallas import tpu as pltpu
from jax.experimental.pallas import tpu_sc as plsc

_D = 32
_B_TOTAL = 4096 * 200
_NC = 2
_NS = 16
_NW = _NC * _NS
_B_PER_W = _B_TOTAL // _NW
_HCH = 4
_ROWS = 128 * _HCH
_N_CH = 200 // _HCH


def _body(doc_hbm, table_hbm, out_hbm, idx_all, idx_ch, rows, buf,
          g0, g1, w0, w1):
    wid = lax.axis_index("s") * _NC + lax.axis_index("c")
    base = wid * _B_PER_W
    pltpu.sync_copy(doc_hbm.at[pl.ds(base, _B_PER_W)], idx_all)

    gsem = (g0, g1)
    wsem = (w0, w1)
    iota = lax.iota(jnp.int32, 16)
    iota200 = iota * 200

    def build_idx(k, j):
        for hp in range(_HCH):
            h = k * _HCH + hp
            for m in range(8):
                pos = iota200 + (m * 3200 + h)
                v = plsc.load_gather(idx_all, [pos])
                idx_ch[j, pl.ds(hp * 128 + m * 16, 16)] = v

    def issue_gather(j):
        return pltpu.async_copy(
            table_hbm.at[idx_ch.at[j]], rows.at[j], gsem[j]
        )

    def wait_gather(j):
        pltpu.make_async_copy(
            table_hbm.at[idx_ch.at[j]], rows.at[j], gsem[j]
        ).wait()

    def transpose_chunk(j):
        rows_j = rows.at[j]

        def s_body(s, carry):
            for hp in range(_HCH):
                for ct in range(4):
                    col = iota * 0 + (8 * ct + s)
                    for lb in range(8):
                        row = iota + (hp * 128 + lb * 16)
                        v = plsc.load_gather(rows_j, [row, col])
                        buf[j, hp, ct, pl.ds(s * 128 + lb * 16, 16)] = v
            return carry

        lax.fori_loop(0, 8, s_body, 0)

    def issue_writes(k, j):
        for hp in range(_HCH):
            for ct in range(4):
                h = k * _HCH + hp
                off = ((h * 4 + ct) * 32 + wid) * 1024
                pltpu.async_copy(
                    buf.at[j, hp, ct], out_hbm.at[pl.ds(off, 1024)], wsem[j]
                )

    def drain_writes(j):
        for _ in range(_HCH * 4):
            pltpu.make_async_copy(
                buf.at[j, 0, 0], out_hbm.at[pl.ds(0, 1024)], wsem[j]
            ).wait()

    build_idx(0, 0)
    issue_gather(0)

    def outer(i, carry):
        for j in range(2):
            k = 2 * i + j
            wait_gather(j)

            @pl.when(k < _N_CH - 1)
            def _():
                build_idx(k + 1, j ^ 1)
                issue_gather(j ^ 1)

            @pl.when(k >= 2)
            def _():
                drain_writes(j)

            transpose_chunk(j)
            issue_writes(k, j)
        return carry

    lax.fori_loop(0, _N_CH // 2, outer, 0)
    drain_writes(0)
    drain_writes(1)


def kernel(doc, table):
    flat = doc.reshape(-1).astype(jnp.int32)
    mesh = plsc.VectorSubcoreMesh(core_axis_name="c", subcore_axis_name="s")
    run = functools.partial(
        pl.kernel,
        mesh=mesh,
        out_type=jax.ShapeDtypeStruct((_B_TOTAL * _D,), jnp.float32),
        scratch_types=[
            pltpu.VMEM((_B_PER_W,), jnp.int32),
            pltpu.VMEM((2, _ROWS), jnp.int32),
            pltpu.VMEM((2, _ROWS, _D), jnp.float32),
            pltpu.VMEM((2, _HCH, 4, 1024), jnp.float32),
            pltpu.SemaphoreType.DMA,
            pltpu.SemaphoreType.DMA,
            pltpu.SemaphoreType.DMA,
            pltpu.SemaphoreType.DMA,
        ],
        compiler_params=pltpu.CompilerParams(
            use_tc_tiling_on_sc=False, needs_layout_passes=False
        ),
    )(_body)
    out1 = run(flat, table)
    out5 = out1.reshape(200, _D // 8, _NW, 8, 128)
    return out5.transpose((2, 4, 0, 1, 3)).reshape(4096, 200, _D)


# speedup vs baseline: 1.6011x; 1.6011x over previous
import functools

import jax
import jax.numpy as jnp
from jax import lax
from jax.experimental import pallas as pl
from jax.experimental.pallas import tpu as pltpu
from jax.experimental.pallas import tpu_sc as plsc

_D = 32
_B_TOTAL = 4096 * 200
_NC = 2
_NS = 16
_NW = _NC * _NS
_B_PER_W = _B_TOTAL // _NW
_HCH = 4
_ROWS = 128 * _HCH
_N_CH = 200 // _HCH


def _body(doc_hbm, table_hbm, out_hbm, idx_all, idx_ch, rows, buf,
          g0, g1, w0, w1):
    wid = lax.axis_index("s") * _NC + lax.axis_index("c")
    base = wid * _B_PER_W
    pltpu.sync_copy(doc_hbm.at[pl.ds(base, _B_PER_W)], idx_all)

    gsem = (g0, g1)
    wsem = (w0, w1)
    iota = lax.iota(jnp.int32, 16)
    iota200 = iota * 200

    def build_idx(k, j):
        for hp in range(_HCH):
            h = k * _HCH + hp
            for m in range(8):
                pos = iota200 + (m * 3200 + h)
                v = plsc.load_gather(idx_all, [pos])
                idx_ch[j, pl.ds(hp * 128 + m * 16, 16)] = v

    def issue_gather(j):
        return pltpu.async_copy(
            table_hbm.at[idx_ch.at[j]], rows.at[j], gsem[j]
        )

    def wait_gather(j):
        pltpu.make_async_copy(
            table_hbm.at[idx_ch.at[j]], rows.at[j], gsem[j]
        ).wait()

    cvecs = {}
    tvecs = {}
    for C0 in (0, 16):
        for d in range(16):
            c = C0 + ((iota + d) & 15)
            cvecs[(C0, d)] = c
            tvecs[(C0, d)] = ((c >> 3) << 10) + ((c & 7) << 7) + iota

    def transpose_chunk(j):
        rows_j = rows.at[j]

        def bb_body(bb, carry):
            b0 = bb * 16
            for hp in range(_HCH):
                buf_hp = buf.at[j, hp]
                r0 = hp * 128 + b0
                for C0 in (0, 16):
                    for d in range(16):
                        v = plsc.load_gather(
                            rows_j, [iota + r0, cvecs[(C0, d)]]
                        )
                        plsc.store_scatter(buf_hp, [tvecs[(C0, d)] + b0], v)
            return carry

        lax.fori_loop(0, 8, bb_body, 0)

    def issue_writes(k, j):
        for hp in range(_HCH):
            for ct in range(4):
                h = k * _HCH + hp
                off = ((h * 4 + ct) * 32 + wid) * 1024
                pltpu.async_copy(
                    buf.at[j, hp, pl.ds(ct * 1024, 1024)],
                    out_hbm.at[pl.ds(off, 1024)],
                    wsem[j],
                )

    def drain_writes(j):
        for _ in range(_HCH * 4):
            pltpu.make_async_copy(
                buf.at[j, 0, pl.ds(0, 1024)],
                out_hbm.at[pl.ds(0, 1024)],
                wsem[j],
            ).wait()

    build_idx(0, 0)
    issue_gather(0)

    def outer(i, carry):
        for j in range(2):
            k = 2 * i + j
            wait_gather(j)

            @pl.when(k < _N_CH - 1)
            def _():
                build_idx(k + 1, j ^ 1)
                issue_gather(j ^ 1)

            @pl.when(k >= 2)
            def _():
                drain_writes(j)

            transpose_chunk(j)
            issue_writes(k, j)
        return carry

    lax.fori_loop(0, _N_CH // 2, outer, 0)
    drain_writes(0)
    drain_writes(1)


def kernel(doc, table):
    flat = doc.reshape(-1).astype(jnp.int32)
    mesh = plsc.VectorSubcoreMesh(core_axis_name="c", subcore_axis_name="s")
    run = functools.partial(
        pl.kernel,
        mesh=mesh,
        out_type=jax.ShapeDtypeStruct((_B_TOTAL * _D,), jnp.float32),
        scratch_types=[
            pltpu.VMEM((_B_PER_W,), jnp.int32),
            pltpu.VMEM((2, _ROWS), jnp.int32),
            pltpu.VMEM((2, _ROWS, _D), jnp.float32),
            pltpu.VMEM((2, _HCH, 4096), jnp.float32),
            pltpu.SemaphoreType.DMA,
            pltpu.SemaphoreType.DMA,
            pltpu.SemaphoreType.DMA,
            pltpu.SemaphoreType.DMA,
        ],
        compiler_params=pltpu.CompilerParams(
            use_tc_tiling_on_sc=False, needs_layout_passes=False
        ),
    )(_body)
    out1 = run(flat, table)
    out5 = out1.reshape(200, _D // 8, _NW, 8, 128)
    return out5.transpose((2, 4, 0, 1, 3)).reshape(4096, 200, _D)


# speedup vs baseline: 1.6784x; 1.0483x over previous
import functools

import jax
import jax.numpy as jnp
from jax import lax
from jax.experimental import pallas as pl
from jax.experimental.pallas import tpu as pltpu
from jax.experimental.pallas import tpu_sc as plsc

_D = 32
_B_TOTAL = 4096 * 200
_NC = 2
_NS = 16
_NW = _NC * _NS
_B_PER_W = _B_TOTAL // _NW
_HCH = 4
_ROWS = 128 * _HCH
_N_CH = 200 // _HCH


def _body(doc_hbm, table_hbm, out_hbm, idx_all, idx_ch, rows, buf,
          g0, g1, w0, w1):
    wid = lax.axis_index("s") * _NC + lax.axis_index("c")
    base = wid * _B_PER_W
    pltpu.sync_copy(doc_hbm.at[pl.ds(base, _B_PER_W)], idx_all)

    gsem = (g0, g1)
    wsem = (w0, w1)
    iota = lax.iota(jnp.int32, 16)
    iota200 = iota * 200

    def build_idx(k, j):
        for hp in range(_HCH):
            h = k * _HCH + hp
            for m in range(8):
                pos = iota200 + (m * 3200 + h)
                v = plsc.load_gather(idx_all, [pos])
                idx_ch[j, pl.ds(hp * 128 + m * 16, 16)] = v

    def issue_gather(j):
        return pltpu.async_copy(
            table_hbm.at[idx_ch.at[j]], rows.at[j], gsem[j]
        )

    def wait_gather(j):
        pltpu.make_async_copy(
            table_hbm.at[idx_ch.at[j]], rows.at[j], gsem[j]
        ).wait()

    cvecs = {}
    tvecs = {}
    for C0 in (0, 16):
        for d in range(16):
            c = C0 + ((iota + d) & 15)
            cvecs[(C0, d)] = c
            tvecs[(C0, d)] = ((c >> 3) << 10) + ((c & 7) << 7) + iota

    def transpose_chunk(j):
        rows_j = rows.at[j]

        def bb_body(bb, carry):
            b0 = bb * 16
            for hp in range(_HCH):
                buf_hp = buf.at[j, hp, pl.ds(b0, 3984)]
                rows_hp = rows_j.at[pl.ds(hp * 128 + b0, 16), :]
                for C0 in (0, 16):
                    for d in range(16):
                        v = plsc.load_gather(
                            rows_hp, [iota, cvecs[(C0, d)]]
                        )
                        plsc.store_scatter(buf_hp, [tvecs[(C0, d)]], v)
            return carry

        lax.fori_loop(0, 8, bb_body, 0)

    def issue_writes(k, j):
        for hp in range(_HCH):
            for ct in range(4):
                h = k * _HCH + hp
                off = ((h * 4 + ct) * 32 + wid) * 1024
                pltpu.async_copy(
                    buf.at[j, hp, pl.ds(ct * 1024, 1024)],
                    out_hbm.at[pl.ds(off, 1024)],
                    wsem[j],
                )

    def drain_writes(j):
        for _ in range(_HCH * 4):
            pltpu.make_async_copy(
                buf.at[j, 0, pl.ds(0, 1024)],
                out_hbm.at[pl.ds(0, 1024)],
                wsem[j],
            ).wait()

    build_idx(0, 0)
    issue_gather(0)

    def outer(i, carry):
        for j in range(2):
            k = 2 * i + j
            wait_gather(j)

            @pl.when(k < _N_CH - 1)
            def _():
                build_idx(k + 1, j ^ 1)
                issue_gather(j ^ 1)

            @pl.when(k >= 2)
            def _():
                drain_writes(j)

            transpose_chunk(j)
            issue_writes(k, j)
        return carry

    lax.fori_loop(0, _N_CH // 2, outer, 0)
    drain_writes(0)
    drain_writes(1)


def kernel(doc, table):
    flat = doc.reshape(-1).astype(jnp.int32)
    mesh = plsc.VectorSubcoreMesh(core_axis_name="c", subcore_axis_name="s")
    run = functools.partial(
        pl.kernel,
        mesh=mesh,
        out_type=jax.ShapeDtypeStruct((_B_TOTAL * _D,), jnp.float32),
        scratch_types=[
            pltpu.VMEM((_B_PER_W,), jnp.int32),
            pltpu.VMEM((2, _ROWS), jnp.int32),
            pltpu.VMEM((2, _ROWS, _D), jnp.float32),
            pltpu.VMEM((2, _HCH, 4096), jnp.float32),
            pltpu.SemaphoreType.DMA,
            pltpu.SemaphoreType.DMA,
            pltpu.SemaphoreType.DMA,
            pltpu.SemaphoreType.DMA,
        ],
        compiler_params=pltpu.CompilerParams(
            use_tc_tiling_on_sc=False, needs_layout_passes=False
        ),
    )(_body)
    out1 = run(flat, table)
    out5 = out1.reshape(200, _D // 8, _NW, 8, 128)
    return out5.transpose((2, 4, 0, 1, 3)).reshape(4096, 200, _D)
